# baseline (speedup 1.0000x reference)
import numpy as np
import jax
import jax.numpy as jnp
from jax import lax
from jax.experimental import pallas as pl

_NP = 6
_NSEG = 8
_NIN = 64
_NOUT = 64
_K = _NP * _NSEG
_LEN = 2.0
_HALF = 1.0
_BT = 512
_G = 1

_X = np.linspace(-1.0, 1.0, _NP).astype(np.float32)
_INVD = np.array(
    [1.0 / np.prod([_X[j] - _X[m] for m in range(_NP) if m != j])
     for j in range(_NP)],
    dtype=np.float32,
)


def _body(x_ref, wg_ref, sw_ref, pw_ref, o_ref):
    xv = x_ref[...].T

    idm = ((xv + _HALF) / _LEN * _NSEG).astype(jnp.int32)
    idm = jnp.minimum(idm, _NSEG - 1)
    idm = jnp.maximum(idm, 0)
    idf = idm.astype(jnp.float32)
    x_min = idf / _NSEG * 2.0 - 1.0
    x_max = (idf + 1.0) / _NSEG * 2.0 - 1.0
    x_in = _LEN * ((xv - x_min) / (x_max - x_min)) - _HALF

    x3 = x_in[:, None, :]
    kidx = lax.broadcasted_iota(jnp.int32, (1, _K, 1), 1)
    kj = kidx % _NP
    ks_f = (kidx // _NP).astype(jnp.float32)
    invd = jnp.zeros((1, _K, 1), jnp.float32)
    for j in range(_NP):
        invd = jnp.where(kj == j, float(_INVD[j]), invd)
    coeff = jnp.broadcast_to(invd, (_NIN, _K, _BT))
    for m in range(_NP):
        coeff = coeff * jnp.where(kj == m, 1.0, x3 - float(_X[m]))
    segmask = idf[:, None, :] == ks_f
    coeff = jnp.where(segmask, coeff, 0.0)

    cg = coeff.reshape(_NIN // _G, _G * _K, _BT)
    sum_acc = jnp.zeros((_NOUT, _BT), jnp.float32)
    prod_acc = jnp.full((_NOUT, _BT), 1.0, jnp.float32)
    for g in range(_NIN // _G):
        a = lax.dot_general(
            wg_ref[g], cg[g],
            (((1,), (0,)), ((), ())),
            preferred_element_type=jnp.float32,
        )
        for t in range(_G):
            at = a[t * _NOUT:(t + 1) * _NOUT]
            sum_acc = sum_acc + at
            prod_acc = prod_acc * at

    res = sum_acc * sw_ref[...] + prod_acc * pw_ref[...]
    o_ref[...] = res.T


def kernel(x, w, sum_w, prod_w):
    batch = x.shape[0]
    wt = jnp.transpose(w, (1, 0, 2))
    if _G == 1:
        wg = wt
    else:
        wt2 = wt.reshape(_NIN // _G, _G, _NOUT, _K)
        eye = jnp.eye(_G, dtype=w.dtype)
        wg = (wt2[:, :, :, None, :] * eye[None, :, None, :, None]).reshape(
            _NIN // _G, _G * _NOUT, _G * _K)

    return pl.pallas_call(
        _body,
        grid=(batch // _BT,),
        in_specs=[
            pl.BlockSpec((_BT, _NIN), lambda t: (t, 0)),
            pl.BlockSpec((_NIN // _G, _G * _NOUT, _G * _K), lambda t: (0, 0, 0)),
            pl.BlockSpec((_NOUT, 1), lambda t: (0, 0)),
            pl.BlockSpec((_NOUT, 1), lambda t: (0, 0)),
        ],
        out_specs=pl.BlockSpec((_BT, _NOUT), lambda t: (t, 0)),
        out_shape=jax.ShapeDtypeStruct((batch, _NOUT), jnp.float32),
    )(x, wg, sum_w.reshape(_NOUT, 1), prod_w.reshape(_NOUT, 1))


# speedup vs baseline: 1.3231x; 1.3231x over previous
import numpy as np
import jax
import jax.numpy as jnp
from jax import lax
from jax.experimental import pallas as pl

_NP = 6
_NSEG = 8
_NIN = 64
_NOUT = 64
_K = _NP * _NSEG
_LEN = 2.0
_HALF = 1.0
_BT = 256

_X = np.linspace(-1.0, 1.0, _NP).astype(np.float32)
_INVD = np.array(
    [1.0 / np.prod([_X[j] - _X[m] for m in range(_NP) if m != j])
     for j in range(_NP)],
    dtype=np.float32,
)
_CR = np.array(
    [[_X[m] for m in range(_NP) if m != j] for j in range(_NP)],
    dtype=np.float32,
).T


def _body(x_ref, wt_ref, sw_ref, pw_ref, o_ref):
    xv = x_ref[...].T

    idm = ((xv + _HALF) / _LEN * _NSEG).astype(jnp.int32)
    idm = jnp.minimum(idm, _NSEG - 1)
    idm = jnp.maximum(idm, 0)
    idf = idm.astype(jnp.float32)
    x_min = idf / _NSEG * 2.0 - 1.0
    x_max = (idf + 1.0) / _NSEG * 2.0 - 1.0
    x_in = _LEN * ((xv - x_min) / (x_max - x_min)) - _HALF

    kidx = lax.broadcasted_iota(jnp.int32, (1, _K, 1), 1)
    kj = kidx % _NP
    ks_f = (kidx // _NP).astype(jnp.float32)
    invd = jnp.zeros((1, _K, 1), jnp.float32)
    for j in range(_NP):
        invd = jnp.where(kj == j, float(_INVD[j]), invd)
    crs = []
    for r in range(_NP - 1):
        c = jnp.zeros((1, _K, 1), jnp.float32)
        for j in range(_NP):
            c = jnp.where(kj == j, float(_CR[r, j]), c)
        crs.append(c)

    x48 = jnp.broadcast_to(x_in[:, None, :], (_NIN, _K, _BT))
    t0 = x48 - crs[0]
    t1 = x48 - crs[1]
    t2 = x48 - crs[2]
    t3 = x48 - crs[3]
    t4 = x48 - crs[4]
    coeff = ((t0 * t1) * (t2 * t3)) * (t4 * invd)
    segmask = idf[:, None, :] == ks_f
    coeff = jnp.where(segmask, coeff, 0.0)

    sum_acc = jnp.zeros((_NOUT, _BT), jnp.float32)
    prod_acc = jnp.full((_NOUT, _BT), 1.0, jnp.float32)
    for i in range(_NIN):
        a = lax.dot_general(
            wt_ref[i], coeff[i],
            (((1,), (0,)), ((), ())),
            preferred_element_type=jnp.float32,
        )
        sum_acc = sum_acc + a
        prod_acc = prod_acc * a

    o_ref[...] = sum_acc.T * sw_ref[...] + prod_acc.T * pw_ref[...]


def kernel(x, w, sum_w, prod_w):
    batch = x.shape[0]
    wt = jnp.transpose(w, (1, 0, 2))
    return pl.pallas_call(
        _body,
        grid=(batch // _BT,),
        in_specs=[
            pl.BlockSpec((_BT, _NIN), lambda t: (t, 0)),
            pl.BlockSpec((_NIN, _NOUT, _K), lambda t: (0, 0, 0)),
            pl.BlockSpec((1, _NOUT), lambda t: (0, 0)),
            pl.BlockSpec((1, _NOUT), lambda t: (0, 0)),
        ],
        out_specs=pl.BlockSpec((_BT, _NOUT), lambda t: (t, 0)),
        out_shape=jax.ShapeDtypeStruct((batch, _NOUT), jnp.float32),
    )(x, wt, sum_w.reshape(1, _NOUT), prod_w.reshape(1, _NOUT))


# speedup vs baseline: 1.5652x; 1.1830x over previous
import numpy as np
import jax
import jax.numpy as jnp
from jax import lax
from jax.experimental import pallas as pl

_NP = 6
_NSEG = 8
_NIN = 64
_NOUT = 64
_K = _NP * _NSEG
_LEN = 2.0
_HALF = 1.0
_BT = 256

_X = np.linspace(-1.0, 1.0, _NP).astype(np.float32)
_INVD = np.array(
    [1.0 / np.prod([_X[j] - _X[m] for m in range(_NP) if m != j])
     for j in range(_NP)],
    dtype=np.float32,
)
_CR = np.array(
    [[_X[m] for m in range(_NP) if m != j] for j in range(_NP)],
    dtype=np.float32,
).T


def _body(x_ref, wt_ref, sw_ref, pw_ref, o_ref):
    xv = x_ref[...].T

    idm = ((xv + _HALF) / _LEN * _NSEG).astype(jnp.int32)
    idm = jnp.minimum(idm, _NSEG - 1)
    idm = jnp.maximum(idm, 0)
    idf = idm.astype(jnp.float32)
    x_min = idf / _NSEG * 2.0 - 1.0
    x_max = (idf + 1.0) / _NSEG * 2.0 - 1.0
    x_in = _LEN * ((xv - x_min) / (x_max - x_min)) - _HALF

    jidx = lax.broadcasted_iota(jnp.int32, (1, _NP, 1), 1)
    invd = jnp.zeros((1, _NP, 1), jnp.float32)
    for j in range(_NP):
        invd = jnp.where(jidx == j, float(_INVD[j]), invd)
    crs = []
    for r in range(_NP - 1):
        c = jnp.zeros((1, _NP, 1), jnp.float32)
        for j in range(_NP):
            c = jnp.where(jidx == j, float(_CR[r, j]), c)
        crs.append(c)
    x6 = jnp.broadcast_to(x_in[:, None, :], (_NIN, _NP, _BT))
    t0 = x6 - crs[0]
    t1 = x6 - crs[1]
    t2 = x6 - crs[2]
    t3 = x6 - crs[3]
    t4 = x6 - crs[4]
    bas = ((t0 * t1) * (t2 * t3)) * (t4 * invd)

    sidx = lax.broadcasted_iota(jnp.int32, (1, _NSEG, 1), 1)
    m8 = jnp.where(idm[:, None, :] == sidx, 1.0, 0.0)

    bx = jnp.repeat(bas, _NSEG, axis=1)
    mx = jnp.concatenate([m8] * _NP, axis=1)
    coeff = bx * mx

    sum_acc = jnp.zeros((_NOUT, _BT), jnp.float32)
    prod_acc = jnp.full((_NOUT, _BT), 1.0, jnp.float32)
    for i in range(_NIN):
        a = lax.dot_general(
            wt_ref[i], coeff[i],
            (((1,), (0,)), ((), ())),
            preferred_element_type=jnp.float32,
        )
        sum_acc = sum_acc + a
        prod_acc = prod_acc * a

    o_ref[...] = sum_acc.T * sw_ref[...] + prod_acc.T * pw_ref[...]


def kernel(x, w, sum_w, prod_w):
    batch = x.shape[0]
    wt = jnp.transpose(
        w.reshape(_NOUT, _NIN, _NSEG, _NP), (1, 0, 3, 2)
    ).reshape(_NIN, _NOUT, _K)
    return pl.pallas_call(
        _body,
        grid=(batch // _BT,),
        in_specs=[
            pl.BlockSpec((_BT, _NIN), lambda t: (t, 0)),
            pl.BlockSpec((_NIN, _NOUT, _K), lambda t: (0, 0, 0)),
            pl.BlockSpec((1, _NOUT), lambda t: (0, 0)),
            pl.BlockSpec((1, _NOUT), lambda t: (0, 0)),
        ],
        out_specs=pl.BlockSpec((_BT, _NOUT), lambda t: (t, 0)),
        out_shape=jax.ShapeDtypeStruct((batch, _NOUT), jnp.float32),
    )(x, wt, sum_w.reshape(1, _NOUT), prod_w.reshape(1, _NOUT))
